# baseline (speedup 1.0000x reference)
import functools
import math

import jax
import jax.numpy as jnp
from jax import lax
from jax.experimental import pallas as pl
from jax.experimental.pallas import tpu as pltpu
from jax.experimental.pallas import tpu_sc as plsc

D = 32
NC, NS, L = 2, 16, 16
NW = NC * NS
W = 128
NBUF = 4
G = 2

_SCALE = math.sqrt(float(D))


@functools.cache
def _make_gather(S, B):
    assert B % (NW * W * NBUF) == 0
    b_per_w = B // NW
    n_chunks = S * (b_per_w // W)
    mesh = plsc.VectorSubcoreMesh(core_axis_name="c", subcore_axis_name="s")

    @functools.partial(
        pl.kernel,
        mesh=mesh,
        out_type=jax.ShapeDtypeStruct((S, D, B), jnp.float32),
        scratch_types=[
            pltpu.VMEM((NBUF, W), jnp.int32),
            pltpu.VMEM((NBUF, W), jnp.int32),
            pltpu.VMEM((NBUF, W, 128), jnp.float32),
            pltpu.VMEM((NBUF, D, W + 1), jnp.float32),
            pltpu.SemaphoreType.DMA((NBUF,)),
            pltpu.SemaphoreType.DMA((NBUF,)),
            pltpu.SemaphoreType.DMA((NBUF,)),
        ],
        compiler_params=pltpu.CompilerParams(
            use_tc_tiling_on_sc=True, needs_layout_passes=False),
    )
    def gather_kernel(emb_lin, idx_t, out3, raw_v, idx4_v, g_v, stg_v,
                      sem_i, sem_g, sem_o):
        wid = lax.axis_index("s") * NC + lax.axis_index("c")
        bstripe = wid * b_per_w

        def idx_copy(ci, sl):
            s = ci // NBUF
            b0 = bstripe + (ci % NBUF) * W
            return pltpu.make_async_copy(
                idx_t.at[s, pl.ds(b0, W)], raw_v.at[sl], sem_i.at[sl])

        def gather_copy(sl):
            return pltpu.make_async_copy(
                emb_lin.at[idx4_v.at[sl]], g_v.at[sl], sem_g.at[sl])

        def out_copy(ci, sl):
            s = ci // NBUF
            b0 = bstripe + (ci % NBUF) * W
            return pltpu.make_async_copy(
                stg_v.at[sl, :, pl.ds(0, W)], out3.at[s, :, pl.ds(b0, W)],
                sem_o.at[sl])

        def prep(sl):
            for u in range(W // L):
                v = raw_v[sl, pl.ds(u * L, L)]
                idx4_v[sl, pl.ds(u * L, L)] = lax.shift_right_logical(v, 2)

        for b in range(NBUF):
            idx_copy(b, b).start()
        for b in range(G):
            idx_copy(b, b).wait()
            prep(b)
            gather_copy(b).start()

        iota = lax.iota(jnp.int32, L)
        r_lo = iota
        r_hi = iota + L

        def chunk_body(ci, carry):
            b = ci % NBUF
            gather_copy(b).wait()

            @pl.when(ci >= NBUF)
            def _drain():
                out_copy(ci - NBUF, b).wait()

            def tgrp(kk, c2):
                vq = (raw_v[b, pl.ds(kk * L, L)] & 3) * D
                for jj in range(L):
                    j = kk * L + jj
                    q = vq[jj]
                    cj = jnp.full((L,), j, jnp.int32)
                    v0 = g_v[b, j, pl.ds(q, L)] * _SCALE
                    v1 = g_v[b, j, pl.ds(q + L, L)] * _SCALE
                    plsc.store_scatter(stg_v.at[b], [r_lo, cj], v0)
                    plsc.store_scatter(stg_v.at[b], [r_hi, cj], v1)
                return c2

            lax.fori_loop(0, W // L, tgrp, 0)

            out_copy(ci, b).start()

            nxt = ci + G
            sp = (b + G) % NBUF

            @pl.when(nxt < n_chunks)
            def _prefetch():
                idx_copy(nxt, sp).wait()
                prep(sp)
                gather_copy(sp).start()

            nxt2 = ci + NBUF

            @pl.when(nxt2 < n_chunks)
            def _refill():
                idx_copy(nxt2, b).start()

            return carry

        lax.fori_loop(0, n_chunks, chunk_body, 0)

        for b in range(NBUF):
            out_copy(n_chunks - NBUF + b, b).wait()

    return gather_kernel


def kernel(inputs, emb):
    n, s = inputs.shape
    emb_lin = emb.reshape(emb.shape[0] * D // 128, 128)
    raw = _make_gather(s, n)(emb_lin, inputs.T)
    return raw.transpose(2, 0, 1)


# speedup vs baseline: 1.3227x; 1.3227x over previous
import functools
import math

import jax
import jax.numpy as jnp
from jax import lax
from jax.experimental import pallas as pl
from jax.experimental.pallas import tpu as pltpu
from jax.experimental.pallas import tpu_sc as plsc

D = 32
NC, NS, L = 2, 16, 16
NW = NC * NS
W = 128
NBUF = 4
G = 2

_SCALE = math.sqrt(float(D))


@functools.cache
def _make_detile(S, B):
    assert B % (NW * W * NBUF) == 0
    b_per_w = B // NW
    kpw = b_per_w // W
    nblk = B // W
    mesh = plsc.VectorSubcoreMesh(core_axis_name="c", subcore_axis_name="s")
    ND = 4
    GD = 2

    @functools.partial(
        pl.kernel,
        mesh=mesh,
        out_type=jax.ShapeDtypeStruct((S * nblk, W), jnp.int32),
        scratch_types=[
            pltpu.VMEM((ND, kpw, W), jnp.int32),
            pltpu.SemaphoreType.DMA((ND,)),
            pltpu.SemaphoreType.DMA((ND,)),
        ],
        compiler_params=pltpu.CompilerParams(use_tc_tiling_on_sc=True),
    )
    def detile_kernel(idx_t, out, buf, sem_in, sem_out):
        wid = lax.axis_index("s") * NC + lax.axis_index("c")
        bstripe = wid * b_per_w

        def in_copy(s, sl, k):
            return pltpu.make_async_copy(
                idx_t.at[s, pl.ds(bstripe + k * W, W)], buf.at[sl, k],
                sem_in.at[sl])

        def out_copy(s, sl):
            return pltpu.make_async_copy(
                buf.at[sl], out.at[pl.ds(s * nblk + wid * kpw, kpw)],
                sem_out.at[sl])

        for s in range(GD):
            for k in range(kpw):
                in_copy(s, s % ND, k).start()

        def body(s, carry):
            sl = s % ND
            for k in range(kpw):
                in_copy(s, sl, k).wait()
            out_copy(s, sl).start()
            nxt = s + GD

            @pl.when(nxt < S)
            def _refill():
                @pl.when(nxt >= ND)
                def _drain():
                    out_copy(nxt - ND, nxt % ND).wait()

                for k in range(kpw):
                    in_copy(nxt, nxt % ND, k).start()

            return carry

        lax.fori_loop(0, S, body, 0)

        for s in range(S - ND, S):
            out_copy(s, s % ND).wait()

    return detile_kernel


@functools.cache
def _make_gather(S, B):
    assert B % (NW * W * NBUF) == 0
    b_per_w = B // NW
    n_chunks = S * (b_per_w // W)
    mesh = plsc.VectorSubcoreMesh(core_axis_name="c", subcore_axis_name="s")

    @functools.partial(
        pl.kernel,
        mesh=mesh,
        out_type=jax.ShapeDtypeStruct((S, D, B), jnp.float32),
        scratch_types=[
            pltpu.VMEM((NBUF, W), jnp.float32),
            pltpu.VMEM((NBUF, W), jnp.int32),
            pltpu.VMEM((NBUF, W, D), jnp.float32),
            pltpu.VMEM((NBUF, D, W + 1), jnp.float32),
            pltpu.SemaphoreType.DMA((NBUF,)),
            pltpu.SemaphoreType.DMA((NBUF,)),
            pltpu.SemaphoreType.DMA((NBUF,)),
        ],
        compiler_params=pltpu.CompilerParams(
            use_tc_tiling_on_sc=False, needs_layout_passes=False),
    )
    def gather_kernel(emb, idx_f, out3, rawf_v, raw_v, g_v, stg_v,
                      sem_i, sem_g, sem_o):
        wid = lax.axis_index("s") * NC + lax.axis_index("c")
        bstripe = wid * b_per_w

        def idx_copy(ci, sl):
            s = ci // NBUF
            b0 = bstripe + (ci % NBUF) * W
            return pltpu.make_async_copy(
                idx_f.at[s, pl.ds(b0, W)], rawf_v.at[sl], sem_i.at[sl])

        def prep(sl):
            for u in range(W // L):
                raw_v[sl, pl.ds(u * L, L)] = plsc.bitcast(
                    rawf_v[sl, pl.ds(u * L, L)], jnp.int32)

        def gather_copy(sl):
            return pltpu.make_async_copy(
                emb.at[raw_v.at[sl]], g_v.at[sl], sem_g.at[sl])

        def out_copy(ci, sl):
            s = ci // NBUF
            b0 = bstripe + (ci % NBUF) * W
            return pltpu.make_async_copy(
                stg_v.at[sl, :, pl.ds(0, W)], out3.at[s, :, pl.ds(b0, W)],
                sem_o.at[sl])

        for b in range(NBUF):
            idx_copy(b, b).start()
        for b in range(G):
            idx_copy(b, b).wait()
            prep(b)
            gather_copy(b).start()

        iota = lax.iota(jnp.int32, L)
        r_lo = iota
        r_hi = iota + L

        def chunk_body(ci, carry):
            b = ci % NBUF
            gather_copy(b).wait()

            @pl.when(ci >= NBUF)
            def _drain():
                out_copy(ci - NBUF, b).wait()

            for j in range(W):
                cj = jnp.full((L,), j, jnp.int32)
                v0 = g_v[b, j, pl.ds(0, L)] * _SCALE
                v1 = g_v[b, j, pl.ds(L, L)] * _SCALE
                plsc.store_scatter(stg_v.at[b], [r_lo, cj], v0)
                plsc.store_scatter(stg_v.at[b], [r_hi, cj], v1)

            out_copy(ci, b).start()

            nxt = ci + G
            sp = (b + G) % NBUF

            @pl.when(nxt < n_chunks)
            def _prefetch():
                idx_copy(nxt, sp).wait()
                prep(sp)
                gather_copy(sp).start()

            nxt2 = ci + NBUF

            @pl.when(nxt2 < n_chunks)
            def _refill():
                idx_copy(nxt2, b).start()

            return carry

        lax.fori_loop(0, n_chunks, chunk_body, 0)

        for b in range(NBUF):
            out_copy(n_chunks - NBUF + b, b).wait()

    return gather_kernel


def kernel(inputs, emb):
    n, s = inputs.shape
    idx_f = lax.bitcast_convert_type(inputs, jnp.float32).T
    raw = _make_gather(s, n)(emb, idx_f)
    return raw.transpose(2, 0, 1)
